# baseline (speedup 1.0000x reference)
import functools
import math

import jax
import jax.numpy as jnp
from jax import lax
from jax.experimental import pallas as pl
from jax.experimental.pallas import tpu as pltpu

B, Q, T, D, G, C = 16, 512, 256, 256, 24, 257
CP = 384
EOS_COEF = 0.1
INV_TEMP = 1.0 / 0.07
ENT0 = math.log(1.0 + 1e-6)
NB2 = 2.0 + 1e-6
ENT2 = -math.log(NB2 + 1e-6) / NB2
NEG_BIG = -1e30


def _dotT(a, b):
    return lax.dot_general(a, b, dimension_numbers=(((0,), (0,)), ((), ())),
                           preferred_element_type=jnp.float32)


def _rowlse(x):
    m = jnp.max(x, axis=1, keepdims=True)
    return jnp.log(jnp.sum(jnp.exp(x - m), axis=1, keepdims=True)) + m


def _one_batch(plog, pm0, pq, pt, pb0, tb0, am, src_row, tgt_row):
    f32 = jnp.float32
    iota_qg = lax.broadcasted_iota(jnp.int32, (Q, G), 0)
    msrc = (iota_qg == src_row).astype(f32)
    iota_gg = lax.broadcasted_iota(jnp.int32, (G, G), 0)
    mtgt = (iota_gg == tgt_row).astype(f32)

    tgt_pm = _dotT(mtgt, pm0)

    lse_c = _rowlse(plog)
    iota_cp = lax.broadcasted_iota(jnp.int32, (1, C), 1)
    last_mask = (iota_cp == (C - 1)).astype(f32)
    pl_last = jnp.sum(plog * last_mask, axis=1, keepdims=True)
    ce_default = EOS_COEF * (Q * ENT0 - jnp.sum(pl_last) + jnp.sum(lse_c))

    m_plog = _dotT(msrc, plog)
    m_lse_c = _rowlse(m_plog)
    m_pl_last = jnp.sum(m_plog * last_mask, axis=1, keepdims=True)
    dot_pm = jnp.sum(m_plog * tgt_pm, axis=1, keepdims=True)
    ent_pm = jnp.sum(jnp.log(tgt_pm + 1e-6) * tgt_pm, axis=1, keepdims=True)
    s_pm = jnp.sum(tgt_pm, axis=1, keepdims=True)
    ce_match = ent_pm - dot_pm + m_lse_c * s_pm
    ce_def_at_m = ENT0 - m_pl_last + m_lse_c
    loss_ce = ce_default + jnp.sum(ce_match - EOS_COEF * ce_def_at_m)

    sbx = _dotT(msrc, pb0)
    dbx = _dotT(mtgt, tb0)
    iota6 = lax.broadcasted_iota(jnp.int32, (1, 6), 1)
    w6 = jnp.where(iota6 < 3, 1.0, 0.2)
    loss_bbox = jnp.sum(jnp.abs(sbx - dbx) * w6)

    def corners(bx):
        cx, cy, cz = bx[:, 0:1], bx[:, 1:2], bx[:, 2:3]
        w = jnp.clip(bx[:, 3:4], 1e-6, None)
        h = jnp.clip(bx[:, 4:5], 1e-6, None)
        d = jnp.clip(bx[:, 5:6], 1e-6, None)
        return (cx - 0.5 * w, cy - 0.5 * h, cz - 0.5 * d,
                cx + 0.5 * w, cy + 0.5 * h, cz + 0.5 * d, w * h * d)

    sx0, sy0, sz0, sx1, sy1, sz1, svol = corners(sbx)
    tx0, ty0, tz0, tx1, ty1, tz1, tvol = corners(dbx)
    ix = jnp.clip(jnp.minimum(sx1, tx1) - jnp.maximum(sx0, tx0), 0.0, None)
    iy = jnp.clip(jnp.minimum(sy1, ty1) - jnp.maximum(sy0, ty0), 0.0, None)
    iz = jnp.clip(jnp.minimum(sz1, tz1) - jnp.maximum(sz0, tz0), 0.0, None)
    inter = ix * iy * iz
    union = svol + tvol - inter
    iou = inter / union
    ex = jnp.clip(jnp.maximum(sx1, tx1) - jnp.minimum(sx0, tx0), 0.0, None)
    ey = jnp.clip(jnp.maximum(sy1, ty1) - jnp.minimum(sy0, ty0), 0.0, None)
    ez = jnp.clip(jnp.maximum(sz1, tz1) - jnp.minimum(sz0, tz0), 0.0, None)
    evol = ex * ey * ez
    giou = iou - (evol - union) / evol
    loss_giou = jnp.sum(1.0 - giou)

    logits = lax.dot_general(pq.astype(jnp.bfloat16), pt.astype(jnp.bfloat16),
                             (((1,), (1,)), ((), ())),
                             preferred_element_type=f32) * INV_TEMP
    m_logits = _dotT(msrc, logits)

    inds = jnp.sum(am, axis=1, keepdims=True) - 1
    c2 = jnp.where(inds == 0, T - 1, inds - 1)
    iota_t = lax.broadcasted_iota(jnp.int32, (1, T), 1)
    col1 = (iota_t == inds)
    colm = (col1 | (iota_t == c2)).astype(f32)
    tmask = jnp.where(col1, 1.0, EOS_COEF)

    lse_t = _rowlse(logits)
    pos_d = -jnp.sum(logits * colm, axis=1, keepdims=True)
    b2t_default = EOS_COEF * jnp.sum(ENT2 + pos_d / NB2 + lse_t)

    tpm_t = tgt_pm[:, 0:T]
    mbool = (tpm_t > 0.0).astype(f32)
    m_lse_t = _rowlse(m_logits)
    m_pos = -jnp.sum(m_logits * mbool, axis=1, keepdims=True)
    m_cnt = jnp.sum(mbool, axis=1, keepdims=True)
    m_nb = m_cnt + 1e-6
    m_ent = -jnp.log(m_nb + 1e-6) / m_nb
    b2t_m = jnp.where(m_cnt > 0.0, m_ent + m_pos / m_nb + m_lse_t, 0.0)
    b2t_d_at_m = ENT2 - jnp.sum(m_logits * colm, axis=1, keepdims=True) / NB2 \
        + m_lse_t
    b2t_total = b2t_default + jnp.sum(b2t_m - EOS_COEF * b2t_d_at_m)

    cmax = jnp.max(logits, axis=0, keepdims=True)
    lse_q = jnp.log(jnp.sum(jnp.exp(logits - cmax), axis=0, keepdims=True)) \
        + cmax
    colsum_all = jnp.sum(logits, axis=0, keepdims=True)
    colsum_m = jnp.sum(m_logits, axis=0, keepdims=True)
    pos_col = -(colsum_all - colsum_m) * colm - jnp.sum(m_logits * mbool,
                                                        axis=0, keepdims=True)
    cnt_col = (Q - G) * colm + jnp.sum(mbool, axis=0, keepdims=True)
    nb_col = cnt_col + 1e-6
    ent_col = -jnp.log(nb_col + 1e-6) / nb_col
    t2b = jnp.where(cnt_col > 0.0, ent_col + pos_col / nb_col + lse_q, 0.0)
    t2b_total = jnp.sum(t2b * tmask)

    contrast = b2t_total + t2b_total

    return loss_ce, loss_bbox, loss_giou, contrast


BB = 4


def _loss_kernel(plog_ref, pm_ref, pq_ref, pt_ref, pb_ref, tb_ref,
                 am_ref, src_ref, tgt_ref, out_ref):
    acc = (0.0, 0.0, 0.0, 0.0)
    for h in range(BB):
        part = _one_batch(plog_ref[0, h], pm_ref[0, h], pq_ref[0, h],
                          pt_ref[0, h], pb_ref[0, h], tb_ref[0, h],
                          am_ref[0, h], src_ref[0, h], tgt_ref[0, h])
        acc = tuple(a + p for a, p in zip(acc, part))

    iota_o = lax.broadcasted_iota(jnp.int32, (1, 128), 1)
    out_ref[0] = (acc[0] * (iota_o == 0) + acc[1] * (iota_o == 1)
                  + acc[2] * (iota_o == 2) + acc[3] * (iota_o == 3)
                  ).astype(jnp.float32)


@jax.jit
def kernel(pred_logits, pred_boxes, proj_tokens, proj_queries, positive_map,
           tgt_boxes, attention_mask, src_idx, tgt_idx):
    NB = B // BB
    plog = pred_logits.reshape(NB, BB, Q, C)
    pm = positive_map.reshape(NB, BB, G, C)
    pq = proj_queries.reshape(NB, BB, Q, D)
    pt = proj_tokens.reshape(NB, BB, T, D)
    pb = pred_boxes.reshape(NB, BB, Q, 6)
    tb = tgt_boxes.reshape(NB, BB, G, 6)
    src3 = src_idx.reshape(NB, BB, 1, G)
    tgt3 = tgt_idx.reshape(NB, BB, 1, G)
    am3 = attention_mask.reshape(NB, BB, 1, T)

    def bs(shape):
        return pl.BlockSpec((1, BB) + shape, lambda b: (b, 0, 0, 0))

    out = pl.pallas_call(
        _loss_kernel,
        grid=(NB,),
        in_specs=[bs((Q, C)), bs((G, C)), bs((Q, D)), bs((T, D)),
                  bs((Q, 6)), bs((G, 6)), bs((1, T)), bs((1, G)), bs((1, G))],
        out_specs=pl.BlockSpec((1, 1, 128), lambda b: (b, 0, 0)),
        out_shape=jax.ShapeDtypeStruct((NB, 1, 128), jnp.float32),
    )(plog, pm, pq, pt, pb, tb, am3, src3, tgt3)

    sums = jnp.sum(out[:, 0, :4], axis=0)
    num_boxes = float(B * G)
    return jnp.stack([sums[0] / num_boxes, sums[1] / num_boxes,
                      sums[2] / num_boxes, sums[3] / 2.0 / num_boxes])


# speedup vs baseline: 1.0237x; 1.0237x over previous
import functools
import math

import jax
import jax.numpy as jnp
from jax import lax
from jax.experimental import pallas as pl
from jax.experimental.pallas import tpu as pltpu

B, Q, T, D, G, C = 16, 512, 256, 256, 24, 257
CP = 384
EOS_COEF = 0.1
INV_TEMP = 1.0 / 0.07
ENT0 = math.log(1.0 + 1e-6)
NB2 = 2.0 + 1e-6
ENT2 = -math.log(NB2 + 1e-6) / NB2
NEG_BIG = -1e30


def _dotT(a, b):
    return lax.dot_general(a, b, dimension_numbers=(((0,), (0,)), ((), ())),
                           preferred_element_type=jnp.float32)


def _rowlse(x):
    m = jnp.max(x, axis=1, keepdims=True)
    return jnp.log(jnp.sum(jnp.exp(x - m), axis=1, keepdims=True)) + m


def _one_batch(plog, pm0, pq, pt, pb0, tb0, am, src_row, tgt_row):
    f32 = jnp.float32
    iota_qg = lax.broadcasted_iota(jnp.int32, (Q, G), 0)
    msrc = (iota_qg == src_row).astype(f32)
    iota_gg = lax.broadcasted_iota(jnp.int32, (G, G), 0)
    mtgt = (iota_gg == tgt_row).astype(f32)

    tgt_pm = _dotT(mtgt, pm0)

    lse_c = _rowlse(plog)
    iota_cp = lax.broadcasted_iota(jnp.int32, (1, C), 1)
    last_mask = (iota_cp == (C - 1)).astype(f32)
    pl_last = jnp.sum(plog * last_mask, axis=1, keepdims=True)
    ce_default = EOS_COEF * (Q * ENT0 - jnp.sum(pl_last) + jnp.sum(lse_c))

    m_plog = _dotT(msrc, plog)
    m_lse_c = _rowlse(m_plog)
    m_pl_last = jnp.sum(m_plog * last_mask, axis=1, keepdims=True)
    dot_pm = jnp.sum(m_plog * tgt_pm, axis=1, keepdims=True)
    ent_pm = jnp.sum(jnp.log(tgt_pm + 1e-6) * tgt_pm, axis=1, keepdims=True)
    s_pm = jnp.sum(tgt_pm, axis=1, keepdims=True)
    ce_match = ent_pm - dot_pm + m_lse_c * s_pm
    ce_def_at_m = ENT0 - m_pl_last + m_lse_c
    loss_ce = ce_default + jnp.sum(ce_match - EOS_COEF * ce_def_at_m)

    sbx = _dotT(msrc, pb0)
    dbx = _dotT(mtgt, tb0)
    iota6 = lax.broadcasted_iota(jnp.int32, (1, 6), 1)
    w6 = jnp.where(iota6 < 3, 1.0, 0.2)
    loss_bbox = jnp.sum(jnp.abs(sbx - dbx) * w6)

    def corners(bx):
        cx, cy, cz = bx[:, 0:1], bx[:, 1:2], bx[:, 2:3]
        w = jnp.clip(bx[:, 3:4], 1e-6, None)
        h = jnp.clip(bx[:, 4:5], 1e-6, None)
        d = jnp.clip(bx[:, 5:6], 1e-6, None)
        return (cx - 0.5 * w, cy - 0.5 * h, cz - 0.5 * d,
                cx + 0.5 * w, cy + 0.5 * h, cz + 0.5 * d, w * h * d)

    sx0, sy0, sz0, sx1, sy1, sz1, svol = corners(sbx)
    tx0, ty0, tz0, tx1, ty1, tz1, tvol = corners(dbx)
    ix = jnp.clip(jnp.minimum(sx1, tx1) - jnp.maximum(sx0, tx0), 0.0, None)
    iy = jnp.clip(jnp.minimum(sy1, ty1) - jnp.maximum(sy0, ty0), 0.0, None)
    iz = jnp.clip(jnp.minimum(sz1, tz1) - jnp.maximum(sz0, tz0), 0.0, None)
    inter = ix * iy * iz
    union = svol + tvol - inter
    iou = inter / union
    ex = jnp.clip(jnp.maximum(sx1, tx1) - jnp.minimum(sx0, tx0), 0.0, None)
    ey = jnp.clip(jnp.maximum(sy1, ty1) - jnp.minimum(sy0, ty0), 0.0, None)
    ez = jnp.clip(jnp.maximum(sz1, tz1) - jnp.minimum(sz0, tz0), 0.0, None)
    evol = ex * ey * ez
    giou = iou - (evol - union) / evol
    loss_giou = jnp.sum(1.0 - giou)

    logits = lax.dot_general(pq.astype(jnp.bfloat16), pt.astype(jnp.bfloat16),
                             (((1,), (1,)), ((), ())),
                             preferred_element_type=f32) * INV_TEMP
    m_logits = _dotT(msrc, logits)

    inds = jnp.sum(am, axis=1, keepdims=True) - 1
    c2 = jnp.where(inds == 0, T - 1, inds - 1)
    iota_t = lax.broadcasted_iota(jnp.int32, (1, T), 1)
    col1 = (iota_t == inds)
    colm = (col1 | (iota_t == c2)).astype(f32)
    tmask = jnp.where(col1, 1.0, EOS_COEF)

    lse_t = _rowlse(logits)
    pos_d = -jnp.sum(logits * colm, axis=1, keepdims=True)
    b2t_default = EOS_COEF * jnp.sum(ENT2 + pos_d / NB2 + lse_t)

    tpm_t = tgt_pm[:, 0:T]
    mbool = (tpm_t > 0.0).astype(f32)
    m_lse_t = _rowlse(m_logits)
    m_pos = -jnp.sum(m_logits * mbool, axis=1, keepdims=True)
    m_cnt = jnp.sum(mbool, axis=1, keepdims=True)
    m_nb = m_cnt + 1e-6
    m_ent = -jnp.log(m_nb + 1e-6) / m_nb
    b2t_m = jnp.where(m_cnt > 0.0, m_ent + m_pos / m_nb + m_lse_t, 0.0)
    b2t_d_at_m = ENT2 - jnp.sum(m_logits * colm, axis=1, keepdims=True) / NB2 \
        + m_lse_t
    b2t_total = b2t_default + jnp.sum(b2t_m - EOS_COEF * b2t_d_at_m)

    cmax = jnp.max(logits, axis=0, keepdims=True)
    lse_q = jnp.log(jnp.sum(jnp.exp(logits - cmax), axis=0, keepdims=True)) \
        + cmax
    colsum_all = jnp.sum(logits, axis=0, keepdims=True)
    colsum_m = jnp.sum(m_logits, axis=0, keepdims=True)
    pos_col = -(colsum_all - colsum_m) * colm - jnp.sum(m_logits * mbool,
                                                        axis=0, keepdims=True)
    cnt_col = (Q - G) * colm + jnp.sum(mbool, axis=0, keepdims=True)
    nb_col = cnt_col + 1e-6
    ent_col = -jnp.log(nb_col + 1e-6) / nb_col
    t2b = jnp.where(cnt_col > 0.0, ent_col + pos_col / nb_col + lse_q, 0.0)
    t2b_total = jnp.sum(t2b * tmask)

    contrast = b2t_total + t2b_total

    return loss_ce, loss_bbox, loss_giou, contrast


BB = 2


def _loss_kernel(plog_ref, pm_ref, pq_ref, pt_ref, pb_ref, tb_ref,
                 am_ref, src_ref, tgt_ref, out_ref):
    acc = (0.0, 0.0, 0.0, 0.0)
    for h in range(BB):
        part = _one_batch(plog_ref[0, h], pm_ref[0, h], pq_ref[0, h],
                          pt_ref[0, h], pb_ref[0, h], tb_ref[0, h],
                          am_ref[0, h], src_ref[0, h], tgt_ref[0, h])
        acc = tuple(a + p for a, p in zip(acc, part))

    iota_o = lax.broadcasted_iota(jnp.int32, (1, 128), 1)
    out_ref[0] = (acc[0] * (iota_o == 0) + acc[1] * (iota_o == 1)
                  + acc[2] * (iota_o == 2) + acc[3] * (iota_o == 3)
                  ).astype(jnp.float32)


@jax.jit
def kernel(pred_logits, pred_boxes, proj_tokens, proj_queries, positive_map,
           tgt_boxes, attention_mask, src_idx, tgt_idx):
    NB = B // BB
    plog = pred_logits.reshape(NB, BB, Q, C)
    pm = positive_map.reshape(NB, BB, G, C)
    pq = proj_queries.reshape(NB, BB, Q, D)
    pt = proj_tokens.reshape(NB, BB, T, D)
    pb = pred_boxes.reshape(NB, BB, Q, 6)
    tb = tgt_boxes.reshape(NB, BB, G, 6)
    src3 = src_idx.reshape(NB, BB, 1, G)
    tgt3 = tgt_idx.reshape(NB, BB, 1, G)
    am3 = attention_mask.reshape(NB, BB, 1, T)

    def bs(shape):
        return pl.BlockSpec((1, BB) + shape, lambda b: (b, 0, 0, 0))

    out = pl.pallas_call(
        _loss_kernel,
        grid=(NB,),
        in_specs=[bs((Q, C)), bs((G, C)), bs((Q, D)), bs((T, D)),
                  bs((Q, 6)), bs((G, 6)), bs((1, T)), bs((1, G)), bs((1, G))],
        out_specs=pl.BlockSpec((1, 1, 128), lambda b: (b, 0, 0)),
        out_shape=jax.ShapeDtypeStruct((NB, 1, 128), jnp.float32),
        compiler_params=pltpu.CompilerParams(
            dimension_semantics=("parallel",)),
    )(plog, pm, pq, pt, pb, tb, am3, src3, tgt3)

    sums = jnp.sum(out[:, 0, :4], axis=0)
    num_boxes = float(B * G)
    return jnp.stack([sums[0] / num_boxes, sums[1] / num_boxes,
                      sums[2] / num_boxes, sums[3] / 2.0 / num_boxes])


# speedup vs baseline: 1.2692x; 1.2399x over previous
import functools
import math

import jax
import jax.numpy as jnp
from jax import lax
from jax.experimental import pallas as pl
from jax.experimental.pallas import tpu as pltpu

B, Q, T, D, G, C = 16, 512, 256, 256, 24, 257
CP = 384
EOS_COEF = 0.1
INV_TEMP = 1.0 / 0.07
ENT0 = math.log(1.0 + 1e-6)
NB2 = 2.0 + 1e-6
ENT2 = -math.log(NB2 + 1e-6) / NB2
NEG_BIG = -1e30


def _dotT(a, b):
    return lax.dot_general(a, b, dimension_numbers=(((0,), (0,)), ((), ())),
                           preferred_element_type=jnp.float32)


def _rowlse(x):
    m = jnp.max(x, axis=1, keepdims=True)
    return jnp.log(jnp.sum(jnp.exp(x - m), axis=1, keepdims=True)) + m


def _one_batch(plog, pm0, pq, pt, pb0, tb0, am, src_row, tgt_row):
    f32 = jnp.float32
    iota_qg = lax.broadcasted_iota(jnp.int32, (Q, G), 0)
    msrc = (iota_qg == src_row).astype(f32)
    iota_gg = lax.broadcasted_iota(jnp.int32, (G, G), 0)
    mtgt = (iota_gg == tgt_row).astype(f32)

    tgt_pm = _dotT(mtgt, pm0)

    lse_c = _rowlse(plog)
    iota_cp = lax.broadcasted_iota(jnp.int32, (1, C), 1)
    last_mask = (iota_cp == (C - 1)).astype(f32)
    pl_last = jnp.sum(plog * last_mask, axis=1, keepdims=True)
    ce_default = EOS_COEF * (Q * ENT0 - jnp.sum(pl_last) + jnp.sum(lse_c))

    m_plog = _dotT(msrc, plog)
    m_lse_c = _rowlse(m_plog)
    m_pl_last = jnp.sum(m_plog * last_mask, axis=1, keepdims=True)
    dot_pm = jnp.sum(m_plog * tgt_pm, axis=1, keepdims=True)
    ent_pm = jnp.sum(jnp.log(tgt_pm + 1e-6) * tgt_pm, axis=1, keepdims=True)
    s_pm = jnp.sum(tgt_pm, axis=1, keepdims=True)
    ce_match = ent_pm - dot_pm + m_lse_c * s_pm
    ce_def_at_m = ENT0 - m_pl_last + m_lse_c
    loss_ce = ce_default + jnp.sum(ce_match - EOS_COEF * ce_def_at_m)

    sbx = _dotT(msrc, pb0)
    dbx = _dotT(mtgt, tb0)
    iota6 = lax.broadcasted_iota(jnp.int32, (1, 6), 1)
    w6 = jnp.where(iota6 < 3, 1.0, 0.2)
    loss_bbox = jnp.sum(jnp.abs(sbx - dbx) * w6)

    def corners(bx):
        cx, cy, cz = bx[:, 0:1], bx[:, 1:2], bx[:, 2:3]
        w = jnp.clip(bx[:, 3:4], 1e-6, None)
        h = jnp.clip(bx[:, 4:5], 1e-6, None)
        d = jnp.clip(bx[:, 5:6], 1e-6, None)
        return (cx - 0.5 * w, cy - 0.5 * h, cz - 0.5 * d,
                cx + 0.5 * w, cy + 0.5 * h, cz + 0.5 * d, w * h * d)

    sx0, sy0, sz0, sx1, sy1, sz1, svol = corners(sbx)
    tx0, ty0, tz0, tx1, ty1, tz1, tvol = corners(dbx)
    ix = jnp.clip(jnp.minimum(sx1, tx1) - jnp.maximum(sx0, tx0), 0.0, None)
    iy = jnp.clip(jnp.minimum(sy1, ty1) - jnp.maximum(sy0, ty0), 0.0, None)
    iz = jnp.clip(jnp.minimum(sz1, tz1) - jnp.maximum(sz0, tz0), 0.0, None)
    inter = ix * iy * iz
    union = svol + tvol - inter
    iou = inter / union
    ex = jnp.clip(jnp.maximum(sx1, tx1) - jnp.minimum(sx0, tx0), 0.0, None)
    ey = jnp.clip(jnp.maximum(sy1, ty1) - jnp.minimum(sy0, ty0), 0.0, None)
    ez = jnp.clip(jnp.maximum(sz1, tz1) - jnp.minimum(sz0, tz0), 0.0, None)
    evol = ex * ey * ez
    giou = iou - (evol - union) / evol
    loss_giou = jnp.sum(1.0 - giou)

    logits = lax.dot_general(pq.astype(jnp.bfloat16), pt.astype(jnp.bfloat16),
                             (((1,), (1,)), ((), ())),
                             preferred_element_type=f32) * INV_TEMP
    m_logits = _dotT(msrc, logits)

    inds = jnp.sum(am, axis=1, keepdims=True) - 1
    c2 = jnp.where(inds == 0, T - 1, inds - 1)
    iota_t = lax.broadcasted_iota(jnp.int32, (1, T), 1)
    col1 = (iota_t == inds)
    colm = (col1 | (iota_t == c2)).astype(f32)
    tmask = jnp.where(col1, 1.0, EOS_COEF)

    lse_t = _rowlse(logits)
    pos_d = -jnp.sum(logits * colm, axis=1, keepdims=True)
    b2t_default = EOS_COEF * jnp.sum(ENT2 + pos_d / NB2 + lse_t)

    tpm_t = tgt_pm[:, 0:T]
    mbool = (tpm_t > 0.0).astype(f32)
    m_lse_t = _rowlse(m_logits)
    m_pos = -jnp.sum(m_logits * mbool, axis=1, keepdims=True)
    m_cnt = jnp.sum(mbool, axis=1, keepdims=True)
    m_nb = m_cnt + 1e-6
    m_ent = -jnp.log(m_nb + 1e-6) / m_nb
    b2t_m = jnp.where(m_cnt > 0.0, m_ent + m_pos / m_nb + m_lse_t, 0.0)
    b2t_d_at_m = ENT2 - jnp.sum(m_logits * colm, axis=1, keepdims=True) / NB2 \
        + m_lse_t
    b2t_total = b2t_default + jnp.sum(b2t_m - EOS_COEF * b2t_d_at_m)

    cmax = jnp.max(logits, axis=0, keepdims=True)
    lse_q = jnp.log(jnp.sum(jnp.exp(logits - cmax), axis=0, keepdims=True)) \
        + cmax
    colsum_all = jnp.sum(logits, axis=0, keepdims=True)
    colsum_m = jnp.sum(m_logits, axis=0, keepdims=True)
    pos_col = -(colsum_all - colsum_m) * colm - jnp.sum(m_logits * mbool,
                                                        axis=0, keepdims=True)
    cnt_col = (Q - G) * colm + jnp.sum(mbool, axis=0, keepdims=True)
    nb_col = cnt_col + 1e-6
    ent_col = -jnp.log(nb_col + 1e-6) / nb_col
    t2b = jnp.where(cnt_col > 0.0, ent_col + pos_col / nb_col + lse_q, 0.0)
    t2b_total = jnp.sum(t2b * tmask)

    contrast = b2t_total + t2b_total

    return loss_ce, loss_bbox, loss_giou, contrast


BB = 2


def _loss_kernel(plog_ref, pm_ref, pq_ref, pt_ref, pb_ref, tb_ref,
                 am_ref, src_ref, tgt_ref, out_ref):
    acc = (0.0, 0.0, 0.0, 0.0)
    base = pl.program_id(0) * BB
    for h in range(BB):
        row = pl.dslice(base + h, 1)
        part = _one_batch(plog_ref[h], pm_ref[h], pq_ref[h],
                          pt_ref[h], pb_ref[h], tb_ref[h],
                          am_ref[row, :], src_ref[row, :],
                          tgt_ref[row, :])
        acc = tuple(a + p for a, p in zip(acc, part))

    iota_o = lax.broadcasted_iota(jnp.int32, (1, 128), 1)
    out_ref[0] = (acc[0] * (iota_o == 0) + acc[1] * (iota_o == 1)
                  + acc[2] * (iota_o == 2) + acc[3] * (iota_o == 3)
                  ).astype(jnp.float32)


@jax.jit
def kernel(pred_logits, pred_boxes, proj_tokens, proj_queries, positive_map,
           tgt_boxes, attention_mask, src_idx, tgt_idx):
    NB = B // BB

    def bs3(shape):
        return pl.BlockSpec((BB,) + shape, lambda b: (b, 0, 0))

    def bs2(n):
        return pl.BlockSpec((B, n), lambda b: (0, 0))

    out = pl.pallas_call(
        _loss_kernel,
        grid=(NB,),
        in_specs=[bs3((Q, C)), bs3((G, C)), bs3((Q, D)), bs3((T, D)),
                  bs3((Q, 6)), bs3((G, 6)), bs2(T), bs2(G), bs2(G)],
        out_specs=pl.BlockSpec((1, 1, 128), lambda b: (b, 0, 0)),
        out_shape=jax.ShapeDtypeStruct((NB, 1, 128), jnp.float32),
        compiler_params=pltpu.CompilerParams(
            dimension_semantics=("parallel",)),
    )(pred_logits, positive_map, proj_queries, proj_tokens,
      pred_boxes, tgt_boxes, attention_mask, src_idx, tgt_idx)

    sums = jnp.sum(out[:, 0, :4], axis=0)
    num_boxes = float(B * G)
    return jnp.stack([sums[0] / num_boxes, sums[1] / num_boxes,
                      sums[2] / num_boxes, sums[3] / 2.0 / num_boxes])
